# baseline (speedup 1.0000x reference)
import jax
import jax.numpy as jnp
import numpy as np
from jax.experimental import pallas as pl
from jax.experimental.pallas import tpu as pltpu

_ROWS = 128
_COLS = 32768
_BLOCK_ROWS = 32
_CHUNK = 128
_UNROLL = 32

_KS0 = 0
_KS1 = 42
_KS2 = _KS0 ^ _KS1 ^ 0x1BD11BDA
_KS = (_KS0, _KS1, _KS2)
_ROT = ((13, 15, 26, 6), (17, 29, 16, 24))

_TINY = np.float32(1.1754943508222875e-38)


def _threefry_bits(x1_in):
    u32 = jnp.uint32
    x0 = x1_in
    x1 = x0 ^ ((x1_in << u32(13)) | (x1_in >> u32(19)))
    first = True
    for j in range(1, 6):
        for r in _ROT[(j - 1) % 2]:
            if first:
                first = False
                continue
            x0 = x0 + x1
            x1 = (x1 << u32(r)) | (x1 >> u32(32 - r))
            x1 = x0 ^ x1
        x0 = x0 + u32(_KS[j % 3])
        x1 = x1 + u32((_KS[(j + 1) % 3] + j) & 0xFFFFFFFF)
    return x0 ^ x1


def _sample_kernel(logits_ref, actions_ref, sum_ref):
    i = pl.program_id(0)
    shape = (_BLOCK_ROWS, _CHUNK)
    row_u = jax.lax.broadcasted_iota(jnp.uint32, shape, 0)
    col_u = jax.lax.broadcasted_iota(jnp.uint32, shape, 1)
    col_i = jax.lax.broadcasted_iota(jnp.int32, shape, 1)
    rowbase = (jnp.uint32(i) * jnp.uint32(_BLOCK_ROWS) + row_u) \
        * jnp.uint32(_COLS) + col_u + jnp.uint32(_KS1)

    def body(c2, carry):
        for sub in range(_UNROLL):
            r_acc, c_acc, l_acc, e_acc = carry
            c = c2 * _UNROLL + sub
            l = logits_ref[:, pl.ds(c * _CHUNK, _CHUNK)]
            bits = _threefry_bits(rowbase + jnp.uint32(c) * jnp.uint32(_CHUNK))

            f = pltpu.bitcast((bits >> jnp.uint32(9)) | jnp.uint32(0x3F800000),
                              jnp.float32) - np.float32(1.0)
            w = -jnp.log(jnp.maximum(f, _TINY))

            e = jnp.exp(l)
            r = e / w
            upd = r > r_acc
            r_acc = jnp.where(upd, r, r_acc)
            c_acc = jnp.where(upd, c, c_acc)
            l_acc = jnp.where(upd, l, l_acc)
            carry = (r_acc, c_acc, l_acc, e_acc + e)
        return carry

    init = (
        jnp.full(shape, np.float32(-1.0)),
        jnp.zeros(shape, jnp.int32),
        jnp.zeros(shape, jnp.float32),
        jnp.zeros(shape, jnp.float32),
    )
    r_acc, c_acc, l_acc, e_acc = jax.lax.fori_loop(
        0, _COLS // (_CHUNK * _UNROLL), body, init)

    r_max = jnp.max(r_acc, axis=1, keepdims=True)
    gidx = c_acc * _CHUNK + col_i
    big = jnp.int32(2**30)
    cand = jnp.where(r_acc == r_max, gidx, big)
    a = jnp.min(cand, axis=1)
    sel = cand == a[:, None]
    l_a = jnp.sum(jnp.where(sel, l_acc, jnp.float32(0.0)), axis=1)
    lse = jnp.log(jnp.sum(e_acc, axis=1))
    partial = jnp.sum(l_a - lse)

    a_row = a[None, :]
    for k in range(_ROWS // _BLOCK_ROWS):
        @pl.when(i == k)
        def _():
            actions_ref[:, k * _BLOCK_ROWS:(k + 1) * _BLOCK_ROWS] = a_row

    @pl.when(i == 0)
    def _():
        sum_ref[:, :] = jnp.zeros((1, 1), jnp.float32)

    sum_ref[:, :] += partial.reshape(1, 1)


def kernel(logits):
    grid = _ROWS // _BLOCK_ROWS
    actions, total = pl.pallas_call(
        _sample_kernel,
        grid=(grid,),
        in_specs=[pl.BlockSpec((_BLOCK_ROWS, _COLS), lambda i: (i, 0))],
        out_specs=[
            pl.BlockSpec((1, _ROWS), lambda i: (0, 0)),
            pl.BlockSpec((1, 1), lambda i: (0, 0)),
        ],
        out_shape=[
            jax.ShapeDtypeStruct((1, _ROWS), jnp.int32),
            jax.ShapeDtypeStruct((1, 1), jnp.float32),
        ],
        compiler_params=pltpu.CompilerParams(
            dimension_semantics=("arbitrary",),
        ),
    )(logits)
    return actions[0], total[0, 0]


# speedup vs baseline: 1.0053x; 1.0053x over previous
import jax
import jax.numpy as jnp
import numpy as np
from jax.experimental import pallas as pl
from jax.experimental.pallas import tpu as pltpu

_ROWS = 128
_COLS = 32768
_BLOCK_ROWS = 32
_CHUNK = 256
_UNROLL = 16

_KS0 = 0
_KS1 = 42
_KS2 = _KS0 ^ _KS1 ^ 0x1BD11BDA
_KS = (_KS0, _KS1, _KS2)
_ROT = ((13, 15, 26, 6), (17, 29, 16, 24))

_TINY = np.float32(1.1754943508222875e-38)


def _threefry_bits(x1_in):
    u32 = jnp.uint32
    x0 = x1_in
    x1 = x0 ^ ((x1_in << u32(13)) | (x1_in >> u32(19)))
    first = True
    for j in range(1, 6):
        for r in _ROT[(j - 1) % 2]:
            if first:
                first = False
                continue
            x0 = x0 + x1
            x1 = (x1 << u32(r)) | (x1 >> u32(32 - r))
            x1 = x0 ^ x1
        x0 = x0 + u32(_KS[j % 3])
        x1 = x1 + u32((_KS[(j + 1) % 3] + j) & 0xFFFFFFFF)
    return x0 ^ x1


def _sample_kernel(logits_ref, actions_ref, sum_ref):
    i = pl.program_id(0)
    shape = (_BLOCK_ROWS, _CHUNK)
    row_u = jax.lax.broadcasted_iota(jnp.uint32, shape, 0)
    col_u = jax.lax.broadcasted_iota(jnp.uint32, shape, 1)
    col_i = jax.lax.broadcasted_iota(jnp.int32, shape, 1)
    rowbase = (jnp.uint32(i) * jnp.uint32(_BLOCK_ROWS) + row_u) \
        * jnp.uint32(_COLS) + col_u + jnp.uint32(_KS1)

    def body(c2, carry):
        for sub in range(_UNROLL):
            r_acc, c_acc, l_acc, e_acc = carry
            c = c2 * _UNROLL + sub
            l = logits_ref[:, pl.ds(c * _CHUNK, _CHUNK)]
            bits = _threefry_bits(rowbase + jnp.uint32(c) * jnp.uint32(_CHUNK))

            f = pltpu.bitcast((bits >> jnp.uint32(9)) | jnp.uint32(0x3F800000),
                              jnp.float32) - np.float32(1.0)
            w = -jnp.log(jnp.maximum(f, _TINY))

            e = jnp.exp(l)
            r = e / w
            upd = r > r_acc
            r_acc = jnp.where(upd, r, r_acc)
            c_acc = jnp.where(upd, c, c_acc)
            l_acc = jnp.where(upd, l, l_acc)
            carry = (r_acc, c_acc, l_acc, e_acc + e)
        return carry

    init = (
        jnp.full(shape, np.float32(-1.0)),
        jnp.zeros(shape, jnp.int32),
        jnp.zeros(shape, jnp.float32),
        jnp.zeros(shape, jnp.float32),
    )
    r_acc, c_acc, l_acc, e_acc = jax.lax.fori_loop(
        0, _COLS // (_CHUNK * _UNROLL), body, init)

    r_max = jnp.max(r_acc, axis=1, keepdims=True)
    gidx = c_acc * _CHUNK + col_i
    big = jnp.int32(2**30)
    cand = jnp.where(r_acc == r_max, gidx, big)
    a = jnp.min(cand, axis=1)
    sel = cand == a[:, None]
    l_a = jnp.sum(jnp.where(sel, l_acc, jnp.float32(0.0)), axis=1)
    lse = jnp.log(jnp.sum(e_acc, axis=1))
    partial = jnp.sum(l_a - lse)

    a_row = a[None, :]
    for k in range(_ROWS // _BLOCK_ROWS):
        @pl.when(i == k)
        def _():
            actions_ref[:, k * _BLOCK_ROWS:(k + 1) * _BLOCK_ROWS] = a_row

    @pl.when(i == 0)
    def _():
        sum_ref[:, :] = jnp.zeros((1, 1), jnp.float32)

    sum_ref[:, :] += partial.reshape(1, 1)


def kernel(logits):
    grid = _ROWS // _BLOCK_ROWS
    actions, total = pl.pallas_call(
        _sample_kernel,
        grid=(grid,),
        in_specs=[pl.BlockSpec((_BLOCK_ROWS, _COLS), lambda i: (i, 0))],
        out_specs=[
            pl.BlockSpec((1, _ROWS), lambda i: (0, 0)),
            pl.BlockSpec((1, 1), lambda i: (0, 0)),
        ],
        out_shape=[
            jax.ShapeDtypeStruct((1, _ROWS), jnp.int32),
            jax.ShapeDtypeStruct((1, 1), jnp.float32),
        ],
        compiler_params=pltpu.CompilerParams(
            dimension_semantics=("arbitrary",),
        ),
    )(logits)
    return actions[0], total[0, 0]


# speedup vs baseline: 1.0280x; 1.0226x over previous
import jax
import jax.numpy as jnp
import numpy as np
from jax.experimental import pallas as pl
from jax.experimental.pallas import tpu as pltpu

_ROWS = 128
_COLS = 32768
_BLOCK_ROWS = 32
_CHUNK = 256
_UNROLL = 16

_KS0 = 0
_KS1 = 42
_KS2 = _KS0 ^ _KS1 ^ 0x1BD11BDA
_KS = (_KS0, _KS1, _KS2)
_ROT = ((13, 15, 26, 6), (17, 29, 16, 24))

_TINY = np.float32(1.1754943508222875e-38)


def _threefry_bits(x1_in):
    u32 = jnp.uint32
    x0 = x1_in
    x1 = x0 ^ ((x1_in << u32(13)) | (x1_in >> u32(19)))
    first = True
    for j in range(1, 6):
        for r in _ROT[(j - 1) % 2]:
            if first:
                first = False
                continue
            x0 = x0 + x1
            x1 = (x1 << u32(r)) | (x1 >> u32(32 - r))
            x1 = x0 ^ x1
        x0 = x0 + u32(_KS[j % 3])
        x1 = x1 + u32((_KS[(j + 1) % 3] + j) & 0xFFFFFFFF)
    return x0 ^ x1


def _sample_kernel(logits_ref, actions_ref, sum_ref):
    i = pl.program_id(0)
    shape = (_BLOCK_ROWS, _CHUNK)
    row_u = jax.lax.broadcasted_iota(jnp.uint32, shape, 0)
    col_u = jax.lax.broadcasted_iota(jnp.uint32, shape, 1)
    col_i = jax.lax.broadcasted_iota(jnp.int32, shape, 1)
    rowbase = (jnp.uint32(i) * jnp.uint32(_BLOCK_ROWS) + row_u) \
        * jnp.uint32(_COLS) + col_u + jnp.uint32(_KS1)

    def body(c2, carry):
        for sub in range(_UNROLL):
            r_acc, c_acc, e_acc = carry
            c = c2 * _UNROLL + sub
            l = logits_ref[:, pl.ds(c * _CHUNK, _CHUNK)]
            bits = _threefry_bits(rowbase + jnp.uint32(c) * jnp.uint32(_CHUNK))

            f = pltpu.bitcast((bits >> jnp.uint32(9)) | jnp.uint32(0x3F800000),
                              jnp.float32) - np.float32(1.0)
            w = -jnp.log(f)

            e = jnp.exp(l)
            r = e / w
            upd = r > r_acc
            r_acc = jnp.where(upd, r, r_acc)
            c_acc = jnp.where(upd, c, c_acc)
            carry = (r_acc, c_acc, e_acc + e)
        return carry

    init = (
        jnp.full(shape, np.float32(-1.0)),
        jnp.zeros(shape, jnp.int32),
        jnp.zeros(shape, jnp.float32),
    )
    r_acc, c_acc, e_acc = jax.lax.fori_loop(
        0, _COLS // (_CHUNK * _UNROLL), body, init)

    r_max = jnp.max(r_acc, axis=1, keepdims=True)
    gidx = c_acc * _CHUNK + col_i
    big = jnp.int32(2**30)
    cand = jnp.where(r_acc == r_max, gidx, big)
    a = jnp.min(cand, axis=1, keepdims=True)

    row_i = jax.lax.broadcasted_iota(jnp.int32, (_BLOCK_ROWS, 1), 0)
    flat_a = (jnp.int32(i) * jnp.int32(_BLOCK_ROWS) + row_i) \
        * jnp.int32(_COLS) + a + jnp.int32(_KS1)
    bits_a = _threefry_bits(flat_a.astype(jnp.uint32))
    f_a = pltpu.bitcast((bits_a >> jnp.uint32(9)) | jnp.uint32(0x3F800000),
                        jnp.float32) - np.float32(1.0)
    w_a = -jnp.log(f_a)
    sum_e = jnp.sum(e_acc, axis=1, keepdims=True)
    partial = jnp.sum(jnp.log(r_max * w_a / sum_e))

    a_row = a[:, 0][None, :]
    for k in range(_ROWS // _BLOCK_ROWS):
        @pl.when(i == k)
        def _():
            actions_ref[:, k * _BLOCK_ROWS:(k + 1) * _BLOCK_ROWS] = a_row

    @pl.when(i == 0)
    def _():
        sum_ref[:, :] = jnp.zeros((1, 1), jnp.float32)

    sum_ref[:, :] += partial.reshape(1, 1)


def kernel(logits):
    grid = _ROWS // _BLOCK_ROWS
    actions, total = pl.pallas_call(
        _sample_kernel,
        grid=(grid,),
        in_specs=[pl.BlockSpec((_BLOCK_ROWS, _COLS), lambda i: (i, 0))],
        out_specs=[
            pl.BlockSpec((1, _ROWS), lambda i: (0, 0)),
            pl.BlockSpec((1, 1), lambda i: (0, 0)),
        ],
        out_shape=[
            jax.ShapeDtypeStruct((1, _ROWS), jnp.int32),
            jax.ShapeDtypeStruct((1, 1), jnp.float32),
        ],
        compiler_params=pltpu.CompilerParams(
            dimension_semantics=("arbitrary",),
        ),
    )(logits)
    return actions[0], total[0, 0]


# speedup vs baseline: 1.0308x; 1.0028x over previous
import jax
import jax.numpy as jnp
import numpy as np
from jax.experimental import pallas as pl
from jax.experimental.pallas import tpu as pltpu

_ROWS = 128
_COLS = 32768
_BLOCK_ROWS = 32
_CHUNK = 256
_UNROLL = 32

_KS0 = 0
_KS1 = 42
_KS2 = _KS0 ^ _KS1 ^ 0x1BD11BDA
_KS = (_KS0, _KS1, _KS2)
_ROT = ((13, 15, 26, 6), (17, 29, 16, 24))

_TINY = np.float32(1.1754943508222875e-38)


def _threefry_bits(x1_in):
    u32 = jnp.uint32
    x0 = x1_in
    x1 = x0 ^ ((x1_in << u32(13)) | (x1_in >> u32(19)))
    first = True
    for j in range(1, 6):
        for r in _ROT[(j - 1) % 2]:
            if first:
                first = False
                continue
            x0 = x0 + x1
            x1 = (x1 << u32(r)) | (x1 >> u32(32 - r))
            x1 = x0 ^ x1
        x0 = x0 + u32(_KS[j % 3])
        x1 = x1 + u32((_KS[(j + 1) % 3] + j) & 0xFFFFFFFF)
    return x0 ^ x1


def _sample_kernel(logits_ref, actions_ref, sum_ref):
    i = pl.program_id(0)
    shape = (_BLOCK_ROWS, _CHUNK)
    row_u = jax.lax.broadcasted_iota(jnp.uint32, shape, 0)
    col_u = jax.lax.broadcasted_iota(jnp.uint32, shape, 1)
    col_i = jax.lax.broadcasted_iota(jnp.int32, shape, 1)
    rowbase = (jnp.uint32(i) * jnp.uint32(_BLOCK_ROWS) + row_u) \
        * jnp.uint32(_COLS) + col_u + jnp.uint32(_KS1)

    def body(c2, carry):
        for sub in range(_UNROLL):
            r_acc, c_acc, e_acc = carry
            c = c2 * _UNROLL + sub
            l = logits_ref[:, pl.ds(c * _CHUNK, _CHUNK)]
            bits = _threefry_bits(rowbase + jnp.uint32(c) * jnp.uint32(_CHUNK))

            f = pltpu.bitcast((bits >> jnp.uint32(9)) | jnp.uint32(0x3F800000),
                              jnp.float32) - np.float32(1.0)
            w = -jnp.log(f)

            e = jnp.exp(l)
            r = e / w
            upd = r > r_acc
            r_acc = jnp.where(upd, r, r_acc)
            c_acc = jnp.where(upd, c, c_acc)
            carry = (r_acc, c_acc, e_acc + e)
        return carry

    init = (
        jnp.full(shape, np.float32(-1.0)),
        jnp.zeros(shape, jnp.int32),
        jnp.zeros(shape, jnp.float32),
    )
    r_acc, c_acc, e_acc = jax.lax.fori_loop(
        0, _COLS // (_CHUNK * _UNROLL), body, init)

    r_max = jnp.max(r_acc, axis=1, keepdims=True)
    gidx = c_acc * _CHUNK + col_i
    big = jnp.int32(2**30)
    cand = jnp.where(r_acc == r_max, gidx, big)
    a = jnp.min(cand, axis=1, keepdims=True)

    row_i = jax.lax.broadcasted_iota(jnp.int32, (_BLOCK_ROWS, 1), 0)
    flat_a = (jnp.int32(i) * jnp.int32(_BLOCK_ROWS) + row_i) \
        * jnp.int32(_COLS) + a + jnp.int32(_KS1)
    bits_a = _threefry_bits(flat_a.astype(jnp.uint32))
    f_a = pltpu.bitcast((bits_a >> jnp.uint32(9)) | jnp.uint32(0x3F800000),
                        jnp.float32) - np.float32(1.0)
    w_a = -jnp.log(f_a)
    sum_e = jnp.sum(e_acc, axis=1, keepdims=True)
    partial = jnp.sum(jnp.log(r_max * w_a / sum_e))

    a_row = a[:, 0][None, :]
    for k in range(_ROWS // _BLOCK_ROWS):
        @pl.when(i == k)
        def _():
            actions_ref[:, k * _BLOCK_ROWS:(k + 1) * _BLOCK_ROWS] = a_row

    @pl.when(i == 0)
    def _():
        sum_ref[:, :] = jnp.zeros((1, 1), jnp.float32)

    sum_ref[:, :] += partial.reshape(1, 1)


def kernel(logits):
    grid = _ROWS // _BLOCK_ROWS
    actions, total = pl.pallas_call(
        _sample_kernel,
        grid=(grid,),
        in_specs=[pl.BlockSpec((_BLOCK_ROWS, _COLS), lambda i: (i, 0))],
        out_specs=[
            pl.BlockSpec((1, _ROWS), lambda i: (0, 0)),
            pl.BlockSpec((1, 1), lambda i: (0, 0)),
        ],
        out_shape=[
            jax.ShapeDtypeStruct((1, _ROWS), jnp.int32),
            jax.ShapeDtypeStruct((1, 1), jnp.float32),
        ],
        compiler_params=pltpu.CompilerParams(
            dimension_semantics=("arbitrary",),
        ),
    )(logits)
    return actions[0], total[0, 0]
